# baseline (speedup 1.0000x reference)
import numpy as np
import jax
import jax.numpy as jnp
from jax import lax
from jax.experimental import pallas as pl
from jax.experimental.pallas import tpu as pltpu

B, S, H, K = 16, 2048, 768, 8192
MAX_SLOTS, STEPS = 10, 4
NS = MAX_SLOTS - 1
NSP = 16
GB = 4
GBF = 2
SBLK = 128
KBLK = K // (S // SBLK)
KBB = 512
SCALE = np.sqrt(float(H))
F32 = jnp.float32

_CP = pltpu.CompilerParams(vmem_limit_bytes=63 * 1024 * 1024)


def _nt(a, b):
    return lax.dot_general(a, b, (((1,), (1,)), ((), ())),
                           preferred_element_type=F32)


def _nn(a, b):
    return lax.dot_general(a, b, (((1,), (0,)), ((), ())),
                           preferred_element_type=F32)


def _tn(a, b):
    return lax.dot_general(a, b, (((0,), (0,)), ((), ())),
                           preferred_element_type=F32)


def _attend(q, p):
    sc = _nt(q, p) / SCALE
    w = jax.nn.softmax(sc, axis=-1)
    return _nn(w, p)


def _init_body(p_ref, l_ref, cb_ref, wi, wq, wk, wv, wsq, sq_ref,
               st_ref, qk_ref, cbsq_ref, wz_ref, sqz_ref, accp, accl):
    i = pl.program_id(0)

    @pl.when(i == 0)
    def _():
        accp[...] = jnp.zeros_like(accp)
        accl[...] = jnp.zeros_like(accl)

    accp[...] += jnp.sum(p_ref[...], axis=1)
    accl[...] += jnp.sum(l_ref[...], axis=1)
    cbb = cb_ref[...]
    cbsq_ref[...] = _nt(jnp.ones((1, H), F32), cbb * cbb)

    @pl.when(i == pl.num_programs(0) - 1)
    def _():
        cat = jnp.concatenate([accp[...], accl[...]], axis=1) * (1.0 / S)
        st = jnp.tanh(_nt(cat, wi[...]))
        st_ref[...] = st
        qk_ref[...] = _nn(_nt(st, wq[...]), wk[...])
        wz_ref[...] = _tn(_nn(wsq[...], wv[...]), wk[...])
        sqz_ref[...] = _nn(_nt(sq_ref[...], wsq[...]), wk[...])


def _step_a_body(gb, p_ref, qk_ref, wz, sqz_ref, aux_ref):
    crs = []
    for j in range(gb):
        p = p_ref[j]
        crs.append(_attend(qk_ref[j], p))
    cr_all = jnp.concatenate(crs, axis=0)
    base = _nn(cr_all, wz[...])
    for j in range(gb):
        qk9 = base[j:j + 1, :] + sqz_ref[...]
        cr9 = _attend(qk9, p_ref[j])
        aux_ref[j] = jnp.concatenate(
            [cr9, crs[j], jnp.zeros((NSP - NS - 1, H), F32)], axis=0)


def _step_b_body(last, refs):
    if last:
        (cr_ref, aux_ref, st_ref, cb_ref, cbsq_ref, spprev_ref, wv, wop,
         sq_ref, wg, bg, ws, bs, wih, whh, bih, bhh, wq, wk,
         nst_ref, qk_ref, ms_ref, sl_ref, sp_ref, cl_ref,
         ctx_s, opp_s, minv_s, ope_s) = refs
    else:
        (cr_ref, aux_ref, st_ref, cb_ref, cbsq_ref, wv, wop,
         sq_ref, wg, bg, ws, bs, wih, whh, bih, bhh, wq, wk,
         nst_ref, qk_ref, ms_ref, sl_ref, sp_ref,
         ctx_s, opp_s, minv_s, ope_s) = refs

    i = pl.program_id(0)

    @pl.when(i == 0)
    def _():
        ctx0 = _nt(cr_ref[...], wv[...])
        ctx_s[...] = ctx0
        opp_s[...] = _nt(ctx0, wop[...])
        minv_s[...] = jnp.full((B, 1), jnp.inf, F32)
        ope_s[...] = jnp.zeros((B, H), F32)

    cbb = cb_ref[...]
    dots = _nt(opp_s[...], cbb)
    dist = cbsq_ref[...] - 2.0 * dots
    bidx = jnp.argmin(dist, axis=1, keepdims=True)
    bmin = jnp.min(dist, axis=1, keepdims=True)
    oh = (lax.broadcasted_iota(jnp.int32, (B, KBB), 1) == bidx).astype(F32)
    cand = _nn(oh, cbb)
    upd = bmin < minv_s[...]
    ope_s[...] = jnp.where(upd, cand, ope_s[...])
    minv_s[...] = jnp.where(upd, bmin, minv_s[...])

    @pl.when(i == pl.num_programs(0) - 1)
    def _():
        _step_b_tail(last, refs)


def _step_b_tail(last, refs):
    if last:
        (cr_ref, aux_ref, st_ref, cb_ref, cbsq_ref, spprev_ref, wv, wop,
         sq_ref, wg, bg, ws, bs, wih, whh, bih, bhh, wq, wk,
         nst_ref, qk_ref, ms_ref, sl_ref, sp_ref, cl_ref,
         ctx_s, opp_s, minv_s, ope_s) = refs
    else:
        (cr_ref, aux_ref, st_ref, cb_ref, cbsq_ref, wv, wop,
         sq_ref, wg, bg, ws, bs, wih, whh, bih, bhh, wq, wk,
         nst_ref, qk_ref, ms_ref, sl_ref, sp_ref,
         ctx_s, opp_s, minv_s, ope_s) = refs
        spprev_ref = cl_ref = None

    ctx = ctx_s[...]
    ope = ope_s[...]
    auxv = _nt(aux_ref[...], wv[...])

    ctxg = _nt(ctx, wg[...])[:, 0:1]
    sqg = _nt(wg[...], sq_ref[...])[0:1, :]
    gl = ctxg + sqg + bg[0, 0]
    probs = jax.nn.sigmoid(gl)
    mask = probs >= 0.5
    any_used = jnp.sum(mask.astype(jnp.int32)) > 0
    top = jnp.argmax(probs, axis=1, keepdims=True)
    lane9 = lax.broadcasted_iota(jnp.int32, (B, NS), 1)
    fb_f = (lane9 == top).astype(F32)
    mask_f = jnp.where(any_used, mask.astype(F32), fb_f)
    cnt = jnp.clip(jnp.sum(mask_f, axis=1, keepdims=True), 1.0, None)
    m16 = jnp.concatenate([mask_f, jnp.zeros((B, NSP - NS), F32)], axis=1)
    tiled = jnp.concatenate([m16] * B, axis=1)
    lane = lax.broadcasted_iota(jnp.int32, (B, B * NSP), 1)
    row = lax.broadcasted_iota(jnp.int32, (B, B * NSP), 0)
    wmat = tiled * ((lane // NSP) == row).astype(F32)
    ssum = _nn(wmat, auxv) / cnt
    msum = jnp.tanh(ope + ssum)
    ms_ref[...] = msum
    stop_in = jnp.concatenate([ctx, msum], axis=1)
    slog = _nt(stop_in, ws[...])[:, 0:1] + bs[0, 0]
    sl_ref[...] = slog
    sprob = jax.nn.sigmoid(slog)
    sp_ref[...] = sprob
    st = st_ref[...]
    gi = _nt(msum, wih[...]) + bih[...]
    gh = _nt(st, whh[...]) + bhh[...]
    r = jax.nn.sigmoid(gi[:, :H] + gh[:, :H])
    z = jax.nn.sigmoid(gi[:, H:2 * H] + gh[:, H:2 * H])
    n = jnp.tanh(gi[:, 2 * H:] + r * gh[:, 2 * H:])
    nst = (1.0 - z) * n + z * st
    nst_ref[...] = nst
    qk_ref[...] = _nn(_nt(nst, wq[...]), wk[...])

    if last:
        sp_all = jnp.concatenate([spprev_ref[...], sprob], axis=1)
        hits = (sp_all >= 0.5).astype(F32)
        firsthit = jnp.argmax(hits, axis=1, keepdims=True)
        nh = jnp.sum(hits, axis=1, keepdims=True)
        cl_ref[...] = jnp.where(nh == 0, jnp.full_like(firsthit, STEPS),
                                firsthit + 1)


def _step_b_mid(*refs):
    return _step_b_body(False, refs)


def _step_b_last(*refs):
    return _step_b_body(True, refs)


def kernel(logic_hidden, prompt_hidden, codebook_emb, W_init, W_q, W_k, W_v,
           slot_queries, W_slot_q, W_op_pre, W_gate, b_gate, W_stop, b_stop,
           W_ih, W_hh, b_ih, b_hh):
    b_gate2 = b_gate.reshape(1, 1)
    b_stop2 = b_stop.reshape(1, 1)
    b_ih2 = b_ih.reshape(1, 3 * H)
    b_hh2 = b_hh.reshape(1, 3 * H)
    wg8 = jnp.concatenate([W_gate, jnp.zeros((7, H), F32)], axis=0)
    ws8 = jnp.concatenate([W_stop, jnp.zeros((7, 2 * H), F32)], axis=0)

    _hh = pl.BlockSpec((H, H), lambda i: (0, 0))
    _sq = pl.BlockSpec((NS, H), lambda i: (0, 0))

    init_call = pl.pallas_call(
        _init_body,
        grid=(S // SBLK,),
        in_specs=[
            pl.BlockSpec((B, SBLK, H), lambda i: (0, i, 0)),
            pl.BlockSpec((B, SBLK, H), lambda i: (0, i, 0)),
            pl.BlockSpec((KBLK, H), lambda i: (i, 0)),
            pl.BlockSpec((H, 2 * H), lambda i: (0, 0)),
            _hh, _hh, _hh, _hh, _sq,
        ],
        out_specs=[
            pl.BlockSpec((B, H), lambda i: (0, 0)),
            pl.BlockSpec((B, H), lambda i: (0, 0)),
            pl.BlockSpec((1, KBLK), lambda i: (0, i)),
            pl.BlockSpec((H, H), lambda i: (0, 0)),
            pl.BlockSpec((NS, H), lambda i: (0, 0)),
        ],
        out_shape=[
            jax.ShapeDtypeStruct((B, H), F32),
            jax.ShapeDtypeStruct((B, H), F32),
            jax.ShapeDtypeStruct((1, K), F32),
            jax.ShapeDtypeStruct((H, H), F32),
            jax.ShapeDtypeStruct((NS, H), F32),
        ],
        scratch_shapes=[pltpu.VMEM((B, H), F32), pltpu.VMEM((B, H), F32)],
        compiler_params=_CP,
    )

    def _make_step_a(gb):
        return pl.pallas_call(
            lambda *refs: _step_a_body(gb, *refs),
            grid=(B // gb,),
            in_specs=[
                pl.BlockSpec((gb, S, H), lambda g: (g, 0, 0)),
                pl.BlockSpec((gb, 1, H), lambda g: (g, 0, 0)),
                _hh, _sq,
            ],
            out_specs=pl.BlockSpec((gb, NSP, H), lambda g: (g, 0, 0)),
            out_shape=jax.ShapeDtypeStruct((B, NSP, H), F32),
            compiler_params=_CP,
        )

    step_a = _make_step_a(GB)

    _bh = jax.ShapeDtypeStruct((B, H), F32)
    _b1 = jax.ShapeDtypeStruct((B, 1), F32)
    _cst = lambda shape: pl.BlockSpec(shape, lambda i: tuple(0 for _ in shape))
    _b_common_pre = [_cst((B, H)), _cst((B * NSP, H)), _cst((B, H)),
                     pl.BlockSpec((KBB, H), lambda i: (i, 0)),
                     pl.BlockSpec((1, KBB), lambda i: (0, i))]
    _b_common_post = [_cst((H, H)), _cst((H, H)), _cst((NS, H)),
                      _cst((8, H)), _cst((1, 1)), _cst((8, 2 * H)),
                      _cst((1, 1)), _cst((3 * H, H)), _cst((3 * H, H)),
                      _cst((1, 3 * H)), _cst((1, 3 * H)),
                      _cst((H, H)), _cst((H, H))]
    _b_scratch = [pltpu.VMEM((B, H), F32), pltpu.VMEM((B, H), F32),
                  pltpu.VMEM((B, 1), F32), pltpu.VMEM((B, H), F32)]
    step_b_mid = pl.pallas_call(
        _step_b_mid,
        grid=(K // KBB,),
        in_specs=_b_common_pre + _b_common_post,
        out_specs=[_cst((B, H)), _cst((B, H)), _cst((B, H)),
                   _cst((B, 1)), _cst((B, 1))],
        out_shape=[_bh, _bh, _bh, _b1, _b1],
        scratch_shapes=_b_scratch,
        compiler_params=_CP,
    )
    step_b_last = pl.pallas_call(
        _step_b_last,
        grid=(K // KBB,),
        in_specs=_b_common_pre + [_cst((B, STEPS - 1))] + _b_common_post,
        out_specs=[_cst((B, H)), _cst((B, H)), _cst((B, H)),
                   _cst((B, 1)), _cst((B, 1)), _cst((B, 1))],
        out_shape=[_bh, _bh, _bh, _b1, _b1,
                   jax.ShapeDtypeStruct((B, 1), jnp.int32)],
        scratch_shapes=_b_scratch,
        compiler_params=_CP,
    )

    state, qk, cbsq, wz, sqz = init_call(
        prompt_hidden, logic_hidden, codebook_emb, W_init, W_q, W_k, W_v,
        W_slot_q, slot_queries)

    stop_logits, stop_probs, summaries = [], [], []
    chain_lengths = None
    for step in range(STEPS):
        aux = step_a(prompt_hidden, qk.reshape(B, 1, H), wz, sqz)
        cr_all = aux[:, NS, :]
        aux_flat = aux.reshape(B * NSP, H)
        if step < STEPS - 1:
            state, qk, msum, slog, sprob = step_b_mid(
                cr_all, aux_flat, state, codebook_emb, cbsq, W_v, W_op_pre,
                slot_queries, wg8, b_gate2, ws8, b_stop2, W_ih, W_hh,
                b_ih2, b_hh2, W_q, W_k)
        else:
            sp_prev = jnp.concatenate(stop_probs, axis=1)
            state, qk, msum, slog, sprob, chain_lengths = step_b_last(
                cr_all, aux_flat, state, codebook_emb, cbsq, sp_prev, W_v,
                W_op_pre, slot_queries, wg8, b_gate2, ws8, b_stop2,
                W_ih, W_hh, b_ih2, b_hh2, W_q, W_k)
        summaries.append(msum)
        stop_logits.append(slog)
        stop_probs.append(sprob)

    stop_logits_t = jnp.concatenate(stop_logits, axis=1)
    stop_probs_t = jnp.concatenate(stop_probs, axis=1)
    summary_stack = jnp.stack(summaries, axis=1)
    return stop_logits_t, stop_probs_t, summary_stack, chain_lengths[:, 0]


# speedup vs baseline: 1.0650x; 1.0650x over previous
import numpy as np
import jax
import jax.numpy as jnp
from jax import lax
from jax.experimental import pallas as pl
from jax.experimental.pallas import tpu as pltpu

B, S, H, K = 16, 2048, 768, 8192
MAX_SLOTS, STEPS = 10, 4
NS = MAX_SLOTS - 1
NSP = 16
GB = 4
GBF = 2
SBLK = 128
KBLK = K // (S // SBLK)
SCALE = np.sqrt(float(H))
F32 = jnp.float32

_CP = pltpu.CompilerParams(vmem_limit_bytes=63 * 1024 * 1024)


def _nt(a, b):
    return lax.dot_general(a, b, (((1,), (1,)), ((), ())),
                           preferred_element_type=F32)


def _nn(a, b):
    return lax.dot_general(a, b, (((1,), (0,)), ((), ())),
                           preferred_element_type=F32)


def _tn(a, b):
    return lax.dot_general(a, b, (((0,), (0,)), ((), ())),
                           preferred_element_type=F32)


def _attend(q, p):
    sc = _nt(q, p) / SCALE
    w = jax.nn.softmax(sc, axis=-1)
    return _nn(w, p)


def _init_body(l_ref, cb_ref, wk, wv, wsq, sq_ref,
               lm_ref, cbsq_ref, wz_ref, sqz_ref, accl):
    i = pl.program_id(0)

    @pl.when(i == 0)
    def _():
        accl[...] = jnp.zeros_like(accl)

    accl[...] += jnp.sum(l_ref[...], axis=1)
    cbb = cb_ref[...]
    cbsq_ref[...] = _nt(jnp.ones((1, H), F32), cbb * cbb)

    @pl.when(i == pl.num_programs(0) - 1)
    def _():
        lm_ref[...] = accl[...] * (1.0 / S)
        wz_ref[...] = _tn(_nn(wsq[...], wv[...]), wk[...])
        sqz_ref[...] = _nn(_nt(sq_ref[...], wsq[...]), wk[...])


def _step_a_common(gb, p_ref, qks, wz, sqz_ref, aux_ref, extra_rows):
    crs = []
    for j in range(gb):
        crs.append(_attend(qks[j], p_ref[j]))
    cr_all = jnp.concatenate(crs, axis=0)
    base = _nn(cr_all, wz[...])
    for j in range(gb):
        qk9 = base[j:j + 1, :] + sqz_ref[...]
        cr9 = _attend(qk9, p_ref[j])
        aux_ref[j] = jnp.concatenate(
            [cr9, crs[j]] + extra_rows[j], axis=0)


def _step_a_body(gb, p_ref, qk_ref, wz, sqz_ref, aux_ref):
    qks = [qk_ref[j] for j in range(gb)]
    zrows = [[jnp.zeros((NSP - NS - 1, H), F32)] for _ in range(gb)]
    _step_a_common(gb, p_ref, qks, wz, sqz_ref, aux_ref, zrows)


def _step_a_first_body(gb, p_ref, lm_ref, wi, wq, wk, wz, sqz_ref, aux_ref):
    sts = []
    for j in range(gb):
        pm = jnp.sum(p_ref[j], axis=0, keepdims=True) * (1.0 / S)
        cat = jnp.concatenate([pm, lm_ref[j]], axis=1)
        sts.append(jnp.tanh(_nt(cat, wi[...])))
    st_all = jnp.concatenate(sts, axis=0)
    qk_all = _nn(_nt(st_all, wq[...]), wk[...])
    qks = [qk_all[j:j + 1, :] for j in range(gb)]
    extra = [[sts[j], jnp.zeros((NSP - NS - 2, H), F32)] for j in range(gb)]
    _step_a_common(gb, p_ref, qks, wz, sqz_ref, aux_ref, extra)


def _step_b_body(last, refs):
    if last:
        (cr_ref, aux_ref, st_ref, cb_ref, cbsq_ref, spprev_ref, wv, wop,
         sq_ref, wg, bg, ws, bs, wih, whh, bih, bhh, wq, wk,
         nst_ref, qk_ref, ms_ref, sl_ref, sp_ref, cl_ref) = refs
    else:
        (cr_ref, aux_ref, st_ref, cb_ref, cbsq_ref, wv, wop,
         sq_ref, wg, bg, ws, bs, wih, whh, bih, bhh, wq, wk,
         nst_ref, qk_ref, ms_ref, sl_ref, sp_ref) = refs

    ctx = _nt(cr_ref[...], wv[...])
    auxv = _nt(aux_ref[...], wv[...])

    cb = cb_ref[...]
    opp = _nt(ctx, wop[...])
    dots = _nt(opp, cb)
    dist = cbsq_ref[...] - 2.0 * dots
    idx = jnp.argmin(dist, axis=1, keepdims=True)
    onehot = (lax.broadcasted_iota(jnp.int32, (B, K), 1) == idx).astype(F32)
    ope = _nn(onehot, cb)

    ctxg = _nt(ctx, wg[...])[:, 0:1]
    sqg = _nt(wg[...], sq_ref[...])[0:1, :]
    gl = ctxg + sqg + bg[0, 0]
    probs = jax.nn.sigmoid(gl)
    mask = probs >= 0.5
    any_used = jnp.sum(mask.astype(jnp.int32)) > 0
    top = jnp.argmax(probs, axis=1, keepdims=True)
    lane9 = lax.broadcasted_iota(jnp.int32, (B, NS), 1)
    fb_f = (lane9 == top).astype(F32)
    mask_f = jnp.where(any_used, mask.astype(F32), fb_f)
    cnt = jnp.clip(jnp.sum(mask_f, axis=1, keepdims=True), 1.0, None)
    m16 = jnp.concatenate([mask_f, jnp.zeros((B, NSP - NS), F32)], axis=1)
    tiled = jnp.concatenate([m16] * B, axis=1)
    lane = lax.broadcasted_iota(jnp.int32, (B, B * NSP), 1)
    row = lax.broadcasted_iota(jnp.int32, (B, B * NSP), 0)
    wmat = tiled * ((lane // NSP) == row).astype(F32)
    ssum = _nn(wmat, auxv) / cnt
    msum = jnp.tanh(ope + ssum)
    ms_ref[...] = msum
    stop_in = jnp.concatenate([ctx, msum], axis=1)
    slog = _nt(stop_in, ws[...])[:, 0:1] + bs[0, 0]
    sl_ref[...] = slog
    sprob = jax.nn.sigmoid(slog)
    sp_ref[...] = sprob
    st = st_ref[...]
    gi = _nt(msum, wih[...]) + bih[...]
    gh = _nt(st, whh[...]) + bhh[...]
    r = jax.nn.sigmoid(gi[:, :H] + gh[:, :H])
    z = jax.nn.sigmoid(gi[:, H:2 * H] + gh[:, H:2 * H])
    n = jnp.tanh(gi[:, 2 * H:] + r * gh[:, 2 * H:])
    nst = (1.0 - z) * n + z * st
    nst_ref[...] = nst
    qk_ref[...] = _nn(_nt(nst, wq[...]), wk[...])

    if last:
        sp_all = jnp.concatenate([spprev_ref[...], sprob], axis=1)
        hits = (sp_all >= 0.5).astype(F32)
        firsthit = jnp.argmax(hits, axis=1, keepdims=True)
        nh = jnp.sum(hits, axis=1, keepdims=True)
        cl_ref[...] = jnp.where(nh == 0, jnp.full_like(firsthit, STEPS),
                                firsthit + 1)


def _step_b_mid(*refs):
    return _step_b_body(False, refs)


def _step_b_last(*refs):
    return _step_b_body(True, refs)


def kernel(logic_hidden, prompt_hidden, codebook_emb, W_init, W_q, W_k, W_v,
           slot_queries, W_slot_q, W_op_pre, W_gate, b_gate, W_stop, b_stop,
           W_ih, W_hh, b_ih, b_hh):
    b_gate2 = b_gate.reshape(1, 1)
    b_stop2 = b_stop.reshape(1, 1)
    b_ih2 = b_ih.reshape(1, 3 * H)
    b_hh2 = b_hh.reshape(1, 3 * H)
    wg8 = jnp.concatenate([W_gate, jnp.zeros((7, H), F32)], axis=0)
    ws8 = jnp.concatenate([W_stop, jnp.zeros((7, 2 * H), F32)], axis=0)

    _hh = pl.BlockSpec((H, H), lambda i: (0, 0))
    _sq = pl.BlockSpec((NS, H), lambda i: (0, 0))

    init_call = pl.pallas_call(
        _init_body,
        grid=(S // SBLK,),
        in_specs=[
            pl.BlockSpec((B, SBLK, H), lambda i: (0, i, 0)),
            pl.BlockSpec((KBLK, H), lambda i: (i, 0)),
            _hh, _hh, _hh, _sq,
        ],
        out_specs=[
            pl.BlockSpec((B, H), lambda i: (0, 0)),
            pl.BlockSpec((1, KBLK), lambda i: (0, i)),
            pl.BlockSpec((H, H), lambda i: (0, 0)),
            pl.BlockSpec((NS, H), lambda i: (0, 0)),
        ],
        out_shape=[
            jax.ShapeDtypeStruct((B, H), F32),
            jax.ShapeDtypeStruct((1, K), F32),
            jax.ShapeDtypeStruct((H, H), F32),
            jax.ShapeDtypeStruct((NS, H), F32),
        ],
        scratch_shapes=[pltpu.VMEM((B, H), F32)],
        compiler_params=_CP,
    )

    step_a = pl.pallas_call(
        lambda *refs: _step_a_body(GB, *refs),
        grid=(B // GB,),
        in_specs=[
            pl.BlockSpec((GB, S, H), lambda g: (g, 0, 0)),
            pl.BlockSpec((GB, 1, H), lambda g: (g, 0, 0)),
            _hh, _sq,
        ],
        out_specs=pl.BlockSpec((GB, NSP, H), lambda g: (g, 0, 0)),
        out_shape=jax.ShapeDtypeStruct((B, NSP, H), F32),
        compiler_params=_CP,
    )

    step_a_first = pl.pallas_call(
        lambda *refs: _step_a_first_body(GBF, *refs),
        grid=(B // GBF,),
        in_specs=[
            pl.BlockSpec((GBF, S, H), lambda g: (g, 0, 0)),
            pl.BlockSpec((GBF, 1, H), lambda g: (g, 0, 0)),
            pl.BlockSpec((H, 2 * H), lambda g: (0, 0)),
            _hh, _hh, _hh, _sq,
        ],
        out_specs=pl.BlockSpec((GBF, NSP, H), lambda g: (g, 0, 0)),
        out_shape=jax.ShapeDtypeStruct((B, NSP, H), F32),
        compiler_params=_CP,
    )

    _bh = jax.ShapeDtypeStruct((B, H), F32)
    _b1 = jax.ShapeDtypeStruct((B, 1), F32)
    step_b_mid = pl.pallas_call(
        _step_b_mid,
        out_shape=[_bh, _bh, _bh, _b1, _b1],
        compiler_params=_CP,
    )
    step_b_last = pl.pallas_call(
        _step_b_last,
        out_shape=[_bh, _bh, _bh, _b1, _b1,
                   jax.ShapeDtypeStruct((B, 1), jnp.int32)],
        compiler_params=_CP,
    )

    lmean, cbsq, wz, sqz = init_call(
        logic_hidden, codebook_emb, W_k, W_v, W_slot_q, slot_queries)

    stop_logits, stop_probs, summaries = [], [], []
    state = qk = chain_lengths = None
    for step in range(STEPS):
        if step == 0:
            aux = step_a_first(prompt_hidden, lmean.reshape(B, 1, H),
                               W_init, W_q, W_k, wz, sqz)
            state = aux[:, NS + 1, :]
        else:
            aux = step_a(prompt_hidden, qk.reshape(B, 1, H), wz, sqz)
        cr_all = aux[:, NS, :]
        aux_flat = aux.reshape(B * NSP, H)
        if step < STEPS - 1:
            state, qk, msum, slog, sprob = step_b_mid(
                cr_all, aux_flat, state, codebook_emb, cbsq, W_v, W_op_pre,
                slot_queries, wg8, b_gate2, ws8, b_stop2, W_ih, W_hh,
                b_ih2, b_hh2, W_q, W_k)
        else:
            sp_prev = jnp.concatenate(stop_probs, axis=1)
            state, qk, msum, slog, sprob, chain_lengths = step_b_last(
                cr_all, aux_flat, state, codebook_emb, cbsq, sp_prev, W_v,
                W_op_pre, slot_queries, wg8, b_gate2, ws8, b_stop2,
                W_ih, W_hh, b_ih2, b_hh2, W_q, W_k)
        summaries.append(msum)
        stop_logits.append(slog)
        stop_probs.append(sprob)

    stop_logits_t = jnp.concatenate(stop_logits, axis=1)
    stop_probs_t = jnp.concatenate(stop_probs, axis=1)
    summary_stack = jnp.stack(summaries, axis=1)
    return stop_logits_t, stop_probs_t, summary_stack, chain_lengths[:, 0]
